# Initial kernel scaffold; baseline (speedup 1.0000x reference)
import jax
import jax.numpy as jnp
from jax.experimental import pallas as pl


def kernel(odm_locs, odm_scores, attention_map, boxes, labels, ignored_regions, priors_cxcy):
    raise NotImplementedError("write your pallas kernel here")


# speedup vs baseline: 6.8207x; 6.8207x over previous
import functools

import jax
import jax.numpy as jnp
from jax.experimental import pallas as pl
from jax.experimental.pallas import tpu as pltpu

N_PRIORS_C = 21420
P_PAD = 21504
BATCH_C = 8
N_OBJ_C = 16
N_IGN_C = 4
N_CLASSES_C = 4
THRESHOLD_C = 0.4
NEG_POS_RATIO_C = 2
THETA_C = 0.1
ATT_HW = 56 * 96


def _pairwise_iou(bx1, by1, bx2, by2, px1, py1, px2, py2):
    lt_x = jnp.maximum(bx1, px1)
    lt_y = jnp.maximum(by1, py1)
    rb_x = jnp.minimum(bx2, px2)
    rb_y = jnp.minimum(by2, py2)
    inter = jnp.clip(rb_x - lt_x, 0.0, None) * jnp.clip(rb_y - lt_y, 0.0, None)
    area_b = (bx2 - bx1) * (by2 - by1)
    area_p = (px2 - px1) * (py2 - py1)
    union = area_b + area_p - inter
    return inter / union


def _loss_kernel(locs_ref, scores_ref, att_ref, boxes_ref, labels_ref,
                 ign_ref, priors_ref, out_ref, acc_ref):
    i = pl.program_id(0)
    nb = pl.num_programs(0)

    @pl.when(i == 0)
    def _init():
        acc_ref[0] = 0.0
        acc_ref[1] = 0.0
        acc_ref[2] = 0.0
        acc_ref[3] = 0.0

    lane = jax.lax.broadcasted_iota(jnp.int32, (1, P_PAD), 1)
    lane_valid = lane < N_PRIORS_C

    pcx = priors_ref[0:1, :]
    pcy = priors_ref[1:2, :]
    pw = priors_ref[2:3, :]
    ph = priors_ref[3:4, :]
    px1 = pcx - pw * 0.5
    py1 = pcy - ph * 0.5
    px2 = pcx + pw * 0.5
    py2 = pcy + ph * 0.5

    b = boxes_ref[0]
    bx1 = b[:, 0:1]
    by1 = b[:, 1:2]
    bx2 = b[:, 2:3]
    by2 = b[:, 3:4]

    ov = _pairwise_iou(bx1, by1, bx2, by2, px1, py1, px2, py2)
    ov = jnp.where(lane_valid, ov, -1.0)

    iota_obj = jax.lax.broadcasted_iota(jnp.int32, (N_OBJ_C, P_PAD), 0)
    iota_pri = jax.lax.broadcasted_iota(jnp.int32, (N_OBJ_C, P_PAD), 1)

    ofp = jnp.max(ov, axis=0, keepdims=True)
    obj_fp = jnp.min(jnp.where(ov == ofp, iota_obj, N_OBJ_C), axis=0,
                     keepdims=True)

    ofo = jnp.max(ov, axis=1, keepdims=True)
    pfo = jnp.min(jnp.where(ov == ofo, iota_pri, P_PAD), axis=1,
                  keepdims=True)
    valid = ofo > 0.0

    c = valid.astype(jnp.int32)
    for s in (1, 2, 4, 8):
        shifted = jnp.concatenate(
            [jnp.zeros((s, 1), jnp.int32), c[: N_OBJ_C - s, :]], axis=0)
        c = c + shifted
    rank = c - 1

    obj_j = jax.lax.broadcasted_iota(jnp.int32, (N_OBJ_C, 1), 0)
    match = pfo == lane
    j_sel = jnp.max(jnp.where(match, obj_j, -1), axis=0, keepdims=True)
    sel = (j_sel == iota_obj) & (j_sel >= 0)
    valid_sel = jnp.max(jnp.where(sel & valid, 1, 0), axis=0,
                        keepdims=True) > 0
    rank_sel = jnp.sum(jnp.where(sel, rank, 0), axis=0, keepdims=True)
    ofp = jnp.where(valid_sel, 1.0, ofp)
    obj_fp = jnp.where(valid_sel, rank_sel, obj_fp)

    onehot = obj_fp == iota_obj
    labels_col = labels_ref[0]
    lab = jnp.max(jnp.where(onehot, labels_col, 0), axis=0, keepdims=True)
    lab = jnp.where(ofp < THRESHOLD_C, 0, lab)
    lab = jnp.where(lane_valid, lab, 0)
    tx1 = jnp.sum(jnp.where(onehot, bx1, 0.0), axis=0, keepdims=True)
    ty1 = jnp.sum(jnp.where(onehot, by1, 0.0), axis=0, keepdims=True)
    tx2 = jnp.sum(jnp.where(onehot, bx2, 0.0), axis=0, keepdims=True)
    ty2 = jnp.sum(jnp.where(onehot, by2, 0.0), axis=0, keepdims=True)

    pos = lab > 0
    posf = pos.astype(jnp.float32)
    n_pos = jnp.sum(posf)

    g = ign_ref[0]
    ign_ov = _pairwise_iou(g[:, 0:1], g[:, 1:2], g[:, 2:3], g[:, 3:4],
                           px1, py1, px2, py2)
    ign = jnp.max(ign_ov, axis=0, keepdims=True) >= THETA_C

    gl = locs_ref[0]
    d_cx = gl[0:1, :] * pw / 10.0 + pcx
    d_cy = gl[1:2, :] * ph / 10.0 + pcy
    d_w = jnp.exp(gl[2:3, :] / 5.0) * pw
    d_h = jnp.exp(gl[3:4, :] / 5.0) * ph
    dx1 = d_cx - d_w * 0.5
    dy1 = d_cy - d_h * 0.5
    dx2 = d_cx + d_w * 0.5
    dy2 = d_cy + d_h * 0.5

    ix1 = jnp.maximum(dx1, tx1)
    iy1 = jnp.maximum(dy1, ty1)
    ix2 = jnp.minimum(dx2, tx2)
    iy2 = jnp.minimum(dy2, ty2)
    inter = jnp.clip(ix2 - ix1, 0.0, None) * jnp.clip(iy2 - iy1, 0.0, None)
    ap = (dx2 - dx1) * (dy2 - dy1)
    at = (tx2 - tx1) * (ty2 - ty1)
    union = ap + at - inter
    iou = inter / (union + 1e-9)
    cxp = (dx1 + dx2) * 0.5
    cyp = (dy1 + dy2) * 0.5
    cxt = (tx1 + tx2) * 0.5
    cyt = (ty1 + ty2) * 0.5
    d2 = (cxp - cxt) ** 2 + (cyp - cyt) ** 2
    ex1 = jnp.minimum(dx1, tx1)
    ey1 = jnp.minimum(dy1, ty1)
    ex2 = jnp.maximum(dx2, tx2)
    ey2 = jnp.maximum(dy2, ty2)
    c2 = (ex2 - ex1) ** 2 + (ey2 - ey1) ** 2 + 1e-7
    diou = 1.0 - iou + d2 / c2
    loc_sum = jnp.sum(diou * posf)

    s = scores_ref[0]
    s0 = s[0:1, :]
    s1 = s[1:2, :]
    s2 = s[2:3, :]
    s3 = s[3:4, :]
    m = jnp.maximum(jnp.maximum(s0, s1), jnp.maximum(s2, s3))
    lse = m + jnp.log(jnp.exp(s0 - m) + jnp.exp(s1 - m)
                      + jnp.exp(s2 - m) + jnp.exp(s3 - m))
    picked = jnp.where(lab == 0, s0, 0.0) + jnp.where(lab == 1, s1, 0.0) \
        + jnp.where(lab == 2, s2, 0.0) + jnp.where(lab == 3, s3, 0.0)
    ce = lse - picked
    conf_pos_sum = jnp.sum(ce * posf)

    neg_mask = jnp.logical_not(pos | ign) & lane_valid
    conf_neg = jnp.where(neg_mask, ce, 0.0)

    k = jnp.minimum(NEG_POS_RATIO_C * n_pos.astype(jnp.int32), N_PRIORS_C)
    bits = jax.lax.bitcast_convert_type(conf_neg, jnp.int32)

    def bs_body(_, carry):
        lo, hi = carry
        mid = lo + (hi - lo + 1) // 2
        cnt = jnp.sum((bits >= mid).astype(jnp.int32))
        take = cnt >= k
        return jnp.where(take, mid, lo), jnp.where(take, hi, mid)

    lo, _ = jax.lax.fori_loop(
        0, 31, bs_body, (jnp.int32(0), jnp.int32(0x7F800000)))
    t = jnp.max(jnp.where(bits <= lo, conf_neg, 0.0))
    cnt_gt = jnp.sum((bits > lo).astype(jnp.int32))
    sum_gt = jnp.sum(jnp.where(bits > lo, conf_neg, 0.0))
    topk = jnp.where(k > 0,
                     sum_gt + (k - cnt_gt).astype(jnp.float32) * t,
                     0.0)

    a = att_ref[0]
    seg = -jnp.sum(jnp.clip(jnp.log(1.0 - a), -100.0, None))

    acc_ref[0] = acc_ref[0] + n_pos
    acc_ref[1] = acc_ref[1] + loc_sum
    acc_ref[2] = acc_ref[2] + conf_pos_sum + topk
    acc_ref[3] = acc_ref[3] + seg

    @pl.when(i == nb - 1)
    def _fin():
        out_ref[0, 0] = (acc_ref[2] + acc_ref[1]) / acc_ref[0] + acc_ref[3]


@jax.jit
def kernel(odm_locs, odm_scores, attention_map, boxes, labels,
           ignored_regions, priors_cxcy):
    B, P, _ = odm_scores.shape
    pad = P_PAD - P

    locs_t = jnp.pad(jnp.transpose(odm_locs, (0, 2, 1)),
                     ((0, 0), (0, 0), (0, pad)))
    scores_t = jnp.pad(jnp.transpose(odm_scores, (0, 2, 1)),
                       ((0, 0), (0, 0), (0, pad)))
    priors_t = jnp.pad(jnp.transpose(priors_cxcy, (1, 0)),
                       ((0, 0), (0, pad)), constant_values=0.25)
    att = attention_map.reshape(B, 1, ATT_HW)
    labels_c = labels.astype(jnp.int32).reshape(B, N_OBJ_C, 1)

    out = pl.pallas_call(
        _loss_kernel,
        grid=(B,),
        in_specs=[
            pl.BlockSpec((1, 4, P_PAD), lambda i: (i, 0, 0)),
            pl.BlockSpec((1, 4, P_PAD), lambda i: (i, 0, 0)),
            pl.BlockSpec((1, 1, ATT_HW), lambda i: (i, 0, 0)),
            pl.BlockSpec((1, N_OBJ_C, 4), lambda i: (i, 0, 0)),
            pl.BlockSpec((1, N_OBJ_C, 1), lambda i: (i, 0, 0)),
            pl.BlockSpec((1, N_IGN_C, 4), lambda i: (i, 0, 0)),
            pl.BlockSpec((4, P_PAD), lambda i: (0, 0)),
        ],
        out_specs=pl.BlockSpec(memory_space=pltpu.SMEM),
        out_shape=jax.ShapeDtypeStruct((1, 1), jnp.float32),
        scratch_shapes=[pltpu.SMEM((4,), jnp.float32)],
    )(locs_t, scores_t, att, boxes, labels_c, ignored_regions, priors_t)
    return out.reshape(())


# speedup vs baseline: 9.1647x; 1.3437x over previous
import functools

import jax
import jax.numpy as jnp
from jax.experimental import pallas as pl
from jax.experimental.pallas import tpu as pltpu

N_PRIORS_C = 21420
P_PAD = 21504
BATCH_C = 8
N_OBJ_C = 16
N_IGN_C = 4
N_CLASSES_C = 4
THRESHOLD_C = 0.4
NEG_POS_RATIO_C = 2
THETA_C = 0.1
ATT_HW = 56 * 96


def _pairwise_iou(bx1, by1, bx2, by2, px1, py1, px2, py2):
    lt_x = jnp.maximum(bx1, px1)
    lt_y = jnp.maximum(by1, py1)
    rb_x = jnp.minimum(bx2, px2)
    rb_y = jnp.minimum(by2, py2)
    inter = jnp.clip(rb_x - lt_x, 0.0, None) * jnp.clip(rb_y - lt_y, 0.0, None)
    area_b = (bx2 - bx1) * (by2 - by1)
    area_p = (px2 - px1) * (py2 - py1)
    union = area_b + area_p - inter
    return inter / union


def _loss_kernel(locs_ref, scores_ref, att_ref, boxes_ref, labels_ref,
                 ign_ref, priors_ref, out_ref, acc_ref, cn_ref, np_ref):
    i = pl.program_id(0)

    @pl.when(i == 0)
    def _init():
        acc_ref[0] = 0.0
        acc_ref[1] = 0.0
        acc_ref[2] = 0.0
        acc_ref[3] = 0.0

    @pl.when(i < BATCH_C)
    def _per_image():
        _image_stage(locs_ref, scores_ref, att_ref, boxes_ref, labels_ref,
                     ign_ref, priors_ref, acc_ref, cn_ref, np_ref, i)

    @pl.when(i == BATCH_C)
    def _final():
        cn = cn_ref[:, 0, :]
        k = jnp.minimum(
            NEG_POS_RATIO_C * np_ref[:, 0, 0:1].astype(jnp.int32),
            N_PRIORS_C)
        bits = jax.lax.bitcast_convert_type(cn, jnp.int32)

        def bs_body(_, carry):
            lo, hi = carry
            mid = lo + (hi - lo + 1) // 2
            cnt = jnp.sum((bits >= mid).astype(jnp.int32), axis=1,
                          keepdims=True)
            take = cnt >= k
            return jnp.where(take, mid, lo), jnp.where(take, hi, mid)

        lo0 = jnp.zeros((BATCH_C, 1), jnp.int32)
        hi0 = jnp.full((BATCH_C, 1), 0x7F800000, jnp.int32)
        lo, _ = jax.lax.fori_loop(0, 31, bs_body, (lo0, hi0))
        t = jnp.max(jnp.where(bits <= lo, cn, 0.0), axis=1, keepdims=True)
        cnt_gt = jnp.sum((bits > lo).astype(jnp.int32), axis=1, keepdims=True)
        sum_gt = jnp.sum(jnp.where(bits > lo, cn, 0.0), axis=1, keepdims=True)
        topk = jnp.where(k > 0,
                         sum_gt + (k - cnt_gt).astype(jnp.float32) * t,
                         0.0)
        topk_total = jnp.sum(topk)
        out_ref[0, 0] = ((acc_ref[2] + topk_total + acc_ref[1]) / acc_ref[0]
                         + acc_ref[3])


def _image_stage(locs_ref, scores_ref, att_ref, boxes_ref, labels_ref,
                 ign_ref, priors_ref, acc_ref, cn_ref, np_ref, i):
    lane = jax.lax.broadcasted_iota(jnp.int32, (1, P_PAD), 1)
    lane_valid = lane < N_PRIORS_C

    pcx = priors_ref[0:1, :]
    pcy = priors_ref[1:2, :]
    pw = priors_ref[2:3, :]
    ph = priors_ref[3:4, :]
    px1 = pcx - pw * 0.5
    py1 = pcy - ph * 0.5
    px2 = pcx + pw * 0.5
    py2 = pcy + ph * 0.5

    b = boxes_ref[0]
    bx1 = b[:, 0:1]
    by1 = b[:, 1:2]
    bx2 = b[:, 2:3]
    by2 = b[:, 3:4]

    ov = _pairwise_iou(bx1, by1, bx2, by2, px1, py1, px2, py2)
    ov = jnp.where(lane_valid, ov, -1.0)

    iota_obj = jax.lax.broadcasted_iota(jnp.int32, (N_OBJ_C, P_PAD), 0)
    iota_pri = jax.lax.broadcasted_iota(jnp.int32, (N_OBJ_C, P_PAD), 1)

    ofp = jnp.max(ov, axis=0, keepdims=True)
    obj_fp = jnp.min(jnp.where(ov == ofp, iota_obj, N_OBJ_C), axis=0,
                     keepdims=True)

    ofo = jnp.max(ov, axis=1, keepdims=True)
    pfo = jnp.min(jnp.where(ov == ofo, iota_pri, P_PAD), axis=1,
                  keepdims=True)
    valid = ofo > 0.0

    c = valid.astype(jnp.int32)
    for s in (1, 2, 4, 8):
        shifted = jnp.concatenate(
            [jnp.zeros((s, 1), jnp.int32), c[: N_OBJ_C - s, :]], axis=0)
        c = c + shifted
    rank = c - 1

    obj_j = jax.lax.broadcasted_iota(jnp.int32, (N_OBJ_C, 1), 0)
    match = pfo == lane
    j_sel = jnp.max(jnp.where(match, obj_j, -1), axis=0, keepdims=True)
    sel = (j_sel == iota_obj) & (j_sel >= 0)
    valid_sel = jnp.max(jnp.where(sel & valid, 1, 0), axis=0,
                        keepdims=True) > 0
    rank_sel = jnp.sum(jnp.where(sel, rank, 0), axis=0, keepdims=True)
    ofp = jnp.where(valid_sel, 1.0, ofp)
    obj_fp = jnp.where(valid_sel, rank_sel, obj_fp)

    onehot = obj_fp == iota_obj
    labels_col = labels_ref[0]
    lab = jnp.max(jnp.where(onehot, labels_col, 0), axis=0, keepdims=True)
    lab = jnp.where(ofp < THRESHOLD_C, 0, lab)
    lab = jnp.where(lane_valid, lab, 0)
    tx1 = jnp.sum(jnp.where(onehot, bx1, 0.0), axis=0, keepdims=True)
    ty1 = jnp.sum(jnp.where(onehot, by1, 0.0), axis=0, keepdims=True)
    tx2 = jnp.sum(jnp.where(onehot, bx2, 0.0), axis=0, keepdims=True)
    ty2 = jnp.sum(jnp.where(onehot, by2, 0.0), axis=0, keepdims=True)

    pos = lab > 0
    posf = pos.astype(jnp.float32)
    n_pos = jnp.sum(posf)

    g = ign_ref[0]
    ign_ov = _pairwise_iou(g[:, 0:1], g[:, 1:2], g[:, 2:3], g[:, 3:4],
                           px1, py1, px2, py2)
    ign = jnp.max(ign_ov, axis=0, keepdims=True) >= THETA_C

    gl = locs_ref[0]
    d_cx = gl[0:1, :] * pw / 10.0 + pcx
    d_cy = gl[1:2, :] * ph / 10.0 + pcy
    d_w = jnp.exp(gl[2:3, :] / 5.0) * pw
    d_h = jnp.exp(gl[3:4, :] / 5.0) * ph
    dx1 = d_cx - d_w * 0.5
    dy1 = d_cy - d_h * 0.5
    dx2 = d_cx + d_w * 0.5
    dy2 = d_cy + d_h * 0.5

    ix1 = jnp.maximum(dx1, tx1)
    iy1 = jnp.maximum(dy1, ty1)
    ix2 = jnp.minimum(dx2, tx2)
    iy2 = jnp.minimum(dy2, ty2)
    inter = jnp.clip(ix2 - ix1, 0.0, None) * jnp.clip(iy2 - iy1, 0.0, None)
    ap = (dx2 - dx1) * (dy2 - dy1)
    at = (tx2 - tx1) * (ty2 - ty1)
    union = ap + at - inter
    iou = inter / (union + 1e-9)
    cxp = (dx1 + dx2) * 0.5
    cyp = (dy1 + dy2) * 0.5
    cxt = (tx1 + tx2) * 0.5
    cyt = (ty1 + ty2) * 0.5
    d2 = (cxp - cxt) ** 2 + (cyp - cyt) ** 2
    ex1 = jnp.minimum(dx1, tx1)
    ey1 = jnp.minimum(dy1, ty1)
    ex2 = jnp.maximum(dx2, tx2)
    ey2 = jnp.maximum(dy2, ty2)
    c2 = (ex2 - ex1) ** 2 + (ey2 - ey1) ** 2 + 1e-7
    diou = 1.0 - iou + d2 / c2
    loc_sum = jnp.sum(diou * posf)

    s = scores_ref[0]
    s0 = s[0:1, :]
    s1 = s[1:2, :]
    s2 = s[2:3, :]
    s3 = s[3:4, :]
    m = jnp.maximum(jnp.maximum(s0, s1), jnp.maximum(s2, s3))
    lse = m + jnp.log(jnp.exp(s0 - m) + jnp.exp(s1 - m)
                      + jnp.exp(s2 - m) + jnp.exp(s3 - m))
    picked = jnp.where(lab == 0, s0, 0.0) + jnp.where(lab == 1, s1, 0.0) \
        + jnp.where(lab == 2, s2, 0.0) + jnp.where(lab == 3, s3, 0.0)
    ce = lse - picked
    conf_pos_sum = jnp.sum(ce * posf)

    neg_mask = jnp.logical_not(pos | ign) & lane_valid
    conf_neg = jnp.where(neg_mask, ce, 0.0)
    cn_ref[i] = conf_neg
    np_ref[i] = jnp.full((1, 128), n_pos, jnp.float32)

    a = att_ref[0]
    seg = -jnp.sum(jnp.clip(jnp.log(1.0 - a), -100.0, None))

    acc_ref[0] = acc_ref[0] + n_pos
    acc_ref[1] = acc_ref[1] + loc_sum
    acc_ref[2] = acc_ref[2] + conf_pos_sum
    acc_ref[3] = acc_ref[3] + seg


@jax.jit
def kernel(odm_locs, odm_scores, attention_map, boxes, labels,
           ignored_regions, priors_cxcy):
    B, P, _ = odm_scores.shape
    pad = P_PAD - P

    locs_t = jnp.pad(jnp.transpose(odm_locs, (0, 2, 1)),
                     ((0, 0), (0, 0), (0, pad)))
    scores_t = jnp.pad(jnp.transpose(odm_scores, (0, 2, 1)),
                       ((0, 0), (0, 0), (0, pad)))
    priors_t = jnp.pad(jnp.transpose(priors_cxcy, (1, 0)),
                       ((0, 0), (0, pad)), constant_values=0.25)
    att = attention_map.reshape(B, 1, ATT_HW)
    labels_c = labels.astype(jnp.int32).reshape(B, N_OBJ_C, 1)

    bm = BATCH_C - 1

    out = pl.pallas_call(
        _loss_kernel,
        grid=(B + 1,),
        in_specs=[
            pl.BlockSpec((1, 4, P_PAD), lambda i: (jnp.minimum(i, bm), 0, 0)),
            pl.BlockSpec((1, 4, P_PAD), lambda i: (jnp.minimum(i, bm), 0, 0)),
            pl.BlockSpec((1, 1, ATT_HW), lambda i: (jnp.minimum(i, bm), 0, 0)),
            pl.BlockSpec((1, N_OBJ_C, 4), lambda i: (jnp.minimum(i, bm), 0, 0)),
            pl.BlockSpec((1, N_OBJ_C, 1), lambda i: (jnp.minimum(i, bm), 0, 0)),
            pl.BlockSpec((1, N_IGN_C, 4), lambda i: (jnp.minimum(i, bm), 0, 0)),
            pl.BlockSpec((4, P_PAD), lambda i: (0, 0)),
        ],
        out_specs=pl.BlockSpec(memory_space=pltpu.SMEM),
        out_shape=jax.ShapeDtypeStruct((1, 1), jnp.float32),
        scratch_shapes=[
            pltpu.SMEM((4,), jnp.float32),
            pltpu.VMEM((BATCH_C, 1, P_PAD), jnp.float32),
            pltpu.VMEM((BATCH_C, 1, 128), jnp.float32),
        ],
    )(locs_t, scores_t, att, boxes, labels_c, ignored_regions, priors_t)
    return out.reshape(())


# speedup vs baseline: 11.8288x; 1.2907x over previous
import functools

import jax
import jax.numpy as jnp
from jax.experimental import pallas as pl
from jax.experimental.pallas import tpu as pltpu

N_PRIORS_C = 21420
P_PAD = 21504
BATCH_C = 8
N_OBJ_C = 16
N_IGN_C = 4
N_CLASSES_C = 4
THRESHOLD_C = 0.4
NEG_POS_RATIO_C = 2
THETA_C = 0.1
ATT_HW = 56 * 96


def _pairwise_iou(bx1, by1, bx2, by2, px1, py1, px2, py2):
    lt_x = jnp.maximum(bx1, px1)
    lt_y = jnp.maximum(by1, py1)
    rb_x = jnp.minimum(bx2, px2)
    rb_y = jnp.minimum(by2, py2)
    inter = jnp.clip(rb_x - lt_x, 0.0, None) * jnp.clip(rb_y - lt_y, 0.0, None)
    area_b = (bx2 - bx1) * (by2 - by1)
    area_p = (px2 - px1) * (py2 - py1)
    union = area_b + area_p - inter
    return inter / union


def _loss_kernel(locs_ref, scores_ref, att_ref, boxes_ref, labels_ref,
                 ign_ref, priors_ref, out_ref, acc_ref, cn_ref, np_ref):
    i = pl.program_id(0)

    @pl.when(i == 0)
    def _init():
        acc_ref[0] = 0.0
        acc_ref[1] = 0.0
        acc_ref[2] = 0.0
        acc_ref[3] = 0.0

    @pl.when(i < BATCH_C)
    def _per_image():
        _image_stage(locs_ref, scores_ref, att_ref, boxes_ref, labels_ref,
                     ign_ref, priors_ref, acc_ref, cn_ref, np_ref, i)

    @pl.when(i == BATCH_C)
    def _final():
        cn = cn_ref[:, 0, :]
        k = jnp.minimum(
            NEG_POS_RATIO_C * np_ref[:, 0, 0:1].astype(jnp.int32),
            N_PRIORS_C)
        bits = jax.lax.bitcast_convert_type(cn, jnp.int32)

        def bs_body(_, carry):
            lo, hi = carry
            mid = lo + (hi - lo + 1) // 2
            cnt = jnp.sum((bits >= mid).astype(jnp.int32), axis=1,
                          keepdims=True)
            take = cnt >= k
            return jnp.where(take, mid, lo), jnp.where(take, hi, mid)

        lo0 = jnp.zeros((BATCH_C, 1), jnp.int32)
        hi0 = jnp.full((BATCH_C, 1), 0x7F800000, jnp.int32)
        lo, _ = jax.lax.fori_loop(0, 31, bs_body, (lo0, hi0))
        t = jnp.max(jnp.where(bits <= lo, cn, 0.0), axis=1, keepdims=True)
        cnt_gt = jnp.sum((bits > lo).astype(jnp.int32), axis=1, keepdims=True)
        sum_gt = jnp.sum(jnp.where(bits > lo, cn, 0.0), axis=1, keepdims=True)
        topk = jnp.where(k > 0,
                         sum_gt + (k - cnt_gt).astype(jnp.float32) * t,
                         0.0)
        topk_total = jnp.sum(topk)
        out_ref[0, 0] = ((acc_ref[2] + topk_total + acc_ref[1]) / acc_ref[0]
                         + acc_ref[3])


def _image_stage(locs_ref, scores_ref, att_ref, boxes_ref, labels_ref,
                 ign_ref, priors_ref, acc_ref, cn_ref, np_ref, i):
    lane = jax.lax.broadcasted_iota(jnp.int32, (1, P_PAD), 1)
    lane_valid = lane < N_PRIORS_C

    pcx = priors_ref[0:1, :]
    pcy = priors_ref[1:2, :]
    pw = priors_ref[2:3, :]
    ph = priors_ref[3:4, :]
    px1 = pcx - pw * 0.5
    py1 = pcy - ph * 0.5
    px2 = pcx + pw * 0.5
    py2 = pcy + ph * 0.5

    b = boxes_ref[0]
    bx1 = b[:, 0:1]
    by1 = b[:, 1:2]
    bx2 = b[:, 2:3]
    by2 = b[:, 3:4]

    ov = _pairwise_iou(bx1, by1, bx2, by2, px1, py1, px2, py2)

    iota_obj = jax.lax.broadcasted_iota(jnp.int32, (N_OBJ_C, P_PAD), 0)
    iota_pri = jax.lax.broadcasted_iota(jnp.int32, (N_OBJ_C, P_PAD), 1)

    ofp = jnp.max(ov, axis=0, keepdims=True)
    obj_fp = jnp.min(jnp.where(ov == ofp, iota_obj, N_OBJ_C), axis=0,
                     keepdims=True)

    ofo = jnp.max(ov, axis=1, keepdims=True)
    pfo = jnp.min(jnp.where(ov == ofo, iota_pri, P_PAD), axis=1,
                  keepdims=True)
    valid = ofo > 0.0

    c = valid.astype(jnp.int32)
    for s in (1, 2, 4, 8):
        shifted = jnp.concatenate(
            [jnp.zeros((s, 1), jnp.int32), c[: N_OBJ_C - s, :]], axis=0)
        c = c + shifted
    rank = c - 1

    obj_j = jax.lax.broadcasted_iota(jnp.int32, (N_OBJ_C, 1), 0)
    match = pfo == lane
    j_sel = jnp.max(jnp.where(match, obj_j, -1), axis=0, keepdims=True)
    onehot2 = (j_sel == iota_obj).astype(jnp.float32)
    w2 = jnp.concatenate([valid.astype(jnp.float32),
                          rank.astype(jnp.float32)], axis=1)
    g2 = jax.lax.dot_general(w2, onehot2, (((0,), (0,)), ((), ())),
                             preferred_element_type=jnp.float32)
    valid_sel = g2[0:1, :] >= 0.5
    ofp = jnp.where(valid_sel, 1.0, ofp)
    obj_f = jnp.where(valid_sel, g2[1:2, :], obj_fp.astype(jnp.float32))

    iota_obj_f = iota_obj.astype(jnp.float32)
    onehot = (obj_f == iota_obj_f).astype(jnp.float32)
    labels_col = labels_ref[0].astype(jnp.float32)
    w5 = jnp.concatenate([labels_col, bx1, by1, bx2, by2], axis=1)
    g5 = jax.lax.dot_general(w5, onehot, (((0,), (0,)), ((), ())),
                             preferred_element_type=jnp.float32)
    lab = jnp.where(ofp < THRESHOLD_C, 0.0, g5[0:1, :])
    tx1 = g5[1:2, :]
    ty1 = g5[2:3, :]
    tx2 = g5[3:4, :]
    ty2 = g5[4:5, :]

    pos = lab > 0.0
    posf = pos.astype(jnp.float32)
    n_pos = jnp.sum(posf)

    g = ign_ref[0]
    gx1 = g[:, 0:1]
    gy1 = g[:, 1:2]
    gx2 = g[:, 2:3]
    gy2 = g[:, 3:4]
    i_x = jnp.clip(jnp.minimum(gx2, px2) - jnp.maximum(gx1, px1), 0.0, None)
    i_y = jnp.clip(jnp.minimum(gy2, py2) - jnp.maximum(gy1, py1), 0.0, None)
    inter_g = i_x * i_y
    area_sum = (gx2 - gx1) * (gy2 - gy1) + (px2 - px1) * (py2 - py1)
    ign = jnp.max(jnp.where(11.0 * inter_g >= area_sum, 1, 0), axis=0,
                  keepdims=True) > 0

    gl = locs_ref[0]
    d_cx = gl[0:1, :] * pw / 10.0 + pcx
    d_cy = gl[1:2, :] * ph / 10.0 + pcy
    d_w = jnp.exp(gl[2:3, :] / 5.0) * pw
    d_h = jnp.exp(gl[3:4, :] / 5.0) * ph
    dx1 = d_cx - d_w * 0.5
    dy1 = d_cy - d_h * 0.5
    dx2 = d_cx + d_w * 0.5
    dy2 = d_cy + d_h * 0.5

    ix1 = jnp.maximum(dx1, tx1)
    iy1 = jnp.maximum(dy1, ty1)
    ix2 = jnp.minimum(dx2, tx2)
    iy2 = jnp.minimum(dy2, ty2)
    inter = jnp.clip(ix2 - ix1, 0.0, None) * jnp.clip(iy2 - iy1, 0.0, None)
    ap = (dx2 - dx1) * (dy2 - dy1)
    at = (tx2 - tx1) * (ty2 - ty1)
    union = ap + at - inter
    iou = inter / (union + 1e-9)
    cxp = (dx1 + dx2) * 0.5
    cyp = (dy1 + dy2) * 0.5
    cxt = (tx1 + tx2) * 0.5
    cyt = (ty1 + ty2) * 0.5
    d2 = (cxp - cxt) ** 2 + (cyp - cyt) ** 2
    ex1 = jnp.minimum(dx1, tx1)
    ey1 = jnp.minimum(dy1, ty1)
    ex2 = jnp.maximum(dx2, tx2)
    ey2 = jnp.maximum(dy2, ty2)
    c2 = (ex2 - ex1) ** 2 + (ey2 - ey1) ** 2 + 1e-7
    diou = 1.0 - iou + d2 / c2
    loc_sum = jnp.sum(diou * posf)

    s = scores_ref[0]
    s0 = s[0:1, :]
    s1 = s[1:2, :]
    s2 = s[2:3, :]
    s3 = s[3:4, :]
    m = jnp.maximum(jnp.maximum(s0, s1), jnp.maximum(s2, s3))
    lse = m + jnp.log(jnp.exp(s0 - m) + jnp.exp(s1 - m)
                      + jnp.exp(s2 - m) + jnp.exp(s3 - m))
    picked = jnp.where(lab == 0.0, s0, 0.0) + jnp.where(lab == 1.0, s1, 0.0) \
        + jnp.where(lab == 2.0, s2, 0.0) + jnp.where(lab == 3.0, s3, 0.0)
    ce = lse - picked
    conf_pos_sum = jnp.sum(ce * posf)

    neg_mask = jnp.logical_not(pos | ign) & lane_valid
    conf_neg = jnp.where(neg_mask, ce, 0.0)
    cn_ref[i] = conf_neg
    np_ref[i] = jnp.full((1, 128), n_pos, jnp.float32)

    a = att_ref[0]
    seg = -jnp.sum(jnp.clip(jnp.log(1.0 - a), -100.0, None))

    acc_ref[0] = acc_ref[0] + n_pos
    acc_ref[1] = acc_ref[1] + loc_sum
    acc_ref[2] = acc_ref[2] + conf_pos_sum
    acc_ref[3] = acc_ref[3] + seg


@jax.jit
def kernel(odm_locs, odm_scores, attention_map, boxes, labels,
           ignored_regions, priors_cxcy):
    B, P, _ = odm_scores.shape
    pad = P_PAD - P

    locs_t = jnp.pad(jnp.transpose(odm_locs, (0, 2, 1)),
                     ((0, 0), (0, 0), (0, pad)))
    scores_t = jnp.pad(jnp.transpose(odm_scores, (0, 2, 1)),
                       ((0, 0), (0, 0), (0, pad)))
    sentinel = jnp.tile(jnp.array([[-9.0], [-9.0], [1.0], [1.0]],
                                  jnp.float32), (1, pad))
    priors_t = jnp.concatenate(
        [jnp.transpose(priors_cxcy, (1, 0)), sentinel], axis=1)
    att = attention_map.reshape(B, 1, ATT_HW)
    labels_c = labels.astype(jnp.int32).reshape(B, N_OBJ_C, 1)

    bm = BATCH_C - 1

    out = pl.pallas_call(
        _loss_kernel,
        grid=(B + 1,),
        in_specs=[
            pl.BlockSpec((1, 4, P_PAD), lambda i: (jnp.minimum(i, bm), 0, 0)),
            pl.BlockSpec((1, 4, P_PAD), lambda i: (jnp.minimum(i, bm), 0, 0)),
            pl.BlockSpec((1, 1, ATT_HW), lambda i: (jnp.minimum(i, bm), 0, 0)),
            pl.BlockSpec((1, N_OBJ_C, 4), lambda i: (jnp.minimum(i, bm), 0, 0)),
            pl.BlockSpec((1, N_OBJ_C, 1), lambda i: (jnp.minimum(i, bm), 0, 0)),
            pl.BlockSpec((1, N_IGN_C, 4), lambda i: (jnp.minimum(i, bm), 0, 0)),
            pl.BlockSpec((4, P_PAD), lambda i: (0, 0)),
        ],
        out_specs=pl.BlockSpec(memory_space=pltpu.SMEM),
        out_shape=jax.ShapeDtypeStruct((1, 1), jnp.float32),
        scratch_shapes=[
            pltpu.SMEM((4,), jnp.float32),
            pltpu.VMEM((BATCH_C, 1, P_PAD), jnp.float32),
            pltpu.VMEM((BATCH_C, 1, 128), jnp.float32),
        ],
    )(locs_t, scores_t, att, boxes, labels_c, ignored_regions, priors_t)
    return out.reshape(())
